# baseline (speedup 1.0000x reference)
import functools

import jax
import jax.numpy as jnp
from jax import lax
from jax.experimental import pallas as pl
from jax.experimental.pallas import tpu as pltpu
from jax.experimental.pallas import tpu_sc as plsc

_ALPHA = 0.1
_BETA = 0.9
_K = 10
_GAMMA = _BETA ** _K + _ALPHA * float(sum(_BETA ** i for i in range(_K)))

_NC = 2
_NS = 16
_NW = _NC * _NS


@functools.lru_cache(maxsize=None)
def _make_deg(N, E):
    EPW = E // _NW
    W = 80
    NWIN = EPW // W
    NPAD = ((N + 128 * _NS - 1) // (128 * _NS)) * (128 * _NS)
    FPT = NPAD // _NS
    mesh = plsc.VectorSubcoreMesh(core_axis_name="c", subcore_axis_name="s")

    @functools.partial(
        pl.kernel,
        out_type=jax.ShapeDtypeStruct((_NC * NPAD,), jnp.float32),
        mesh=mesh,
        scratch_types=(
            [pltpu.VMEM((W,), jnp.int32) for _ in range(4)]
            + [pltpu.VMEM((W,), jnp.float32),
               pltpu.VMEM((FPT,), jnp.float32),
               pltpu.VMEM_SHARED((NPAD,), jnp.float32)]
            + [pltpu.SemaphoreType.DMA for _ in range(9)]
        ),
    )
    def deg_kernel(row_hbm, out_hbm, *sc):
        P = 4
        idxw = sc[0:P]
        oneb = sc[P]
        zb = sc[P + 1]
        dacc = sc[P + 2]
        isem = sc[P + 3:2 * P + 3]
        ssem = sc[2 * P + 3:3 * P + 3]
        csem = sc[3 * P + 3]

        c = lax.axis_index("c")
        s = lax.axis_index("s")
        one = jnp.ones((16,), jnp.float32)
        zero = jnp.zeros((16,), jnp.float32)
        for q in range(W // 16):
            oneb[pl.ds(q * 16, 16)] = one
        for q in range(FPT // 16):
            zb[pl.ds(q * 16, 16)] = zero
        pltpu.sync_copy(zb, dacc.at[pl.ds(s * FPT, FPT)])
        plsc.subcore_barrier()
        base = (c * _NS + s) * EPW

        def stage(t, p):
            pltpu.async_copy(row_hbm.at[pl.ds(base + t * W, W)], idxw[p],
                             isem[p])

        def fire_scatter(p):
            pltpu.make_async_copy(row_hbm.at[pl.ds(base, W)], idxw[p],
                                  isem[p]).wait()
            pltpu.async_copy(oneb, dacc.at[idxw[p]], ssem[p], add=True)

        def wait_scatter(p):
            pltpu.make_async_copy(oneb, dacc.at[idxw[p]], ssem[p]).wait()

        stage(0, 0)

        def chunk(jj, carry):
            for p_off in range(P):
                t = 1 + jj * P + p_off
                p_a = (1 + p_off) % P
                p_b = p_off % P

                @pl.when(jnp.logical_and(t >= P, t - P < NWIN))
                def _():
                    wait_scatter(p_a)

                @pl.when(t < NWIN)
                def _():
                    stage(t, p_a)

                @pl.when(t - 1 < NWIN)
                def _():
                    fire_scatter(p_b)
            return carry

        nchunks = NWIN // P
        lax.fori_loop(0, nchunks, chunk, 0)
        t_last = nchunks * P
        for t in range(t_last + 1, NWIN + 1):
            wait_scatter((t - P) % P)
            fire_scatter((t - 1) % P)
        w_hi = (NWIN - P) if NWIN > t_last else (t_last - P)
        for w in range(w_hi + 1, NWIN):
            wait_scatter(w % P)

        plsc.subcore_barrier()
        pltpu.sync_copy(dacc.at[pl.ds(s * FPT, FPT)],
                        out_hbm.at[pl.ds(c * NPAD + s * FPT, FPT)])

    return deg_kernel, NPAD


@functools.lru_cache(maxsize=None)
def _make_spmm(N, D, E):
    EPW = E // _NW
    W = 80
    NWIN = EPW // W
    NPAD = ((N + 32 * _NS - 1) // (32 * _NS)) * (32 * _NS)
    RPT = NPAD // _NS
    CH = 32
    NCH = RPT // CH
    P = 4
    mesh = plsc.VectorSubcoreMesh(core_axis_name="c", subcore_axis_name="s")

    scratch = (
        [pltpu.VMEM((2 * W,), jnp.int32) for _ in range(P)]
        + [pltpu.VMEM((W, D), jnp.float32) for _ in range(P)]
        + [pltpu.VMEM((CH, D), jnp.float32)]
        + [pltpu.VMEM_SHARED((NPAD, D), jnp.float32)]
        + [pltpu.SemaphoreType.DMA for _ in range(3 * P + 1)]
    )

    @functools.partial(
        pl.kernel,
        out_type=jax.ShapeDtypeStruct((_NC, NPAD, D), jnp.float32),
        mesh=mesh,
        scratch_types=scratch,
    )
    def spmm_kernel(u_hbm, eint_hbm, out_hbm, *sc):
        idxw = sc[0:P]
        gbuf = sc[P:2 * P]
        zbuf = sc[2 * P]
        acc = sc[2 * P + 1]
        isem = sc[2 * P + 2:3 * P + 2]
        gsem = sc[3 * P + 2:4 * P + 2]
        ssem = sc[4 * P + 2:5 * P + 2]
        csem = sc[5 * P + 2]

        c = lax.axis_index("c")
        s = lax.axis_index("s")
        zero = jnp.zeros((16,), jnp.float32)
        for r in range(CH):
            for q in range(D // 16):
                zbuf[r, pl.ds(q * 16, 16)] = zero
        r0 = s * RPT
        for t in range(NCH):
            pltpu.async_copy(zbuf, acc.at[pl.ds(r0 + t * CH, CH)], csem)
        wbase = (c * _NS + s) * NWIN
        for t in range(NCH):
            pltpu.make_async_copy(zbuf, acc.at[pl.ds(r0, CH)], csem).wait()
        plsc.subcore_barrier()

        def stage_idx(t, p):
            off = (wbase + t) * (2 * W)
            pltpu.async_copy(eint_hbm.at[pl.ds(off, 2 * W)], idxw[p], isem[p])

        def fire_gather(t, p):
            pltpu.make_async_copy(eint_hbm.at[pl.ds(0, 2 * W)], idxw[p],
                                  isem[p]).wait()
            pltpu.async_copy(u_hbm.at[idxw[p].at[pl.ds(0, W)]], gbuf[p],
                             gsem[p])

        def fire_scatter(p):
            pltpu.make_async_copy(u_hbm.at[idxw[p].at[pl.ds(0, W)]], gbuf[p],
                                  gsem[p]).wait()
            pltpu.async_copy(gbuf[p], acc.at[idxw[p].at[pl.ds(W, W)]],
                             ssem[p], add=True)

        def wait_scatter(p):
            pltpu.make_async_copy(gbuf[p], acc.at[idxw[p].at[pl.ds(W, W)]],
                                  ssem[p]).wait()

        stage_idx(0, 0)
        stage_idx(1, 1)
        fire_gather(0, 0)

        def chunk(jj, carry):
            for p_off in range(P):
                t = 2 + jj * P + p_off

                @pl.when(jnp.logical_and(t >= 4, t - 4 < NWIN))
                def _():
                    wait_scatter((2 + p_off) % P)

                @pl.when(t < NWIN)
                def _():
                    stage_idx(t, (2 + p_off) % P)

                @pl.when(t <= NWIN)
                def _():
                    fire_gather(t - 1, (1 + p_off) % P)

                @pl.when(t - 2 < NWIN)
                def _():
                    fire_scatter(p_off % P)
            return carry

        nchunks = (NWIN - 1 + P - 1) // P
        lax.fori_loop(0, nchunks, chunk, 0)
        t_last = 2 + nchunks * P - 1
        done = min(nchunks * P, NWIN)
        for t in range(done, NWIN):
            fire_scatter(t % P)
        waited = min(t_last - 4, NWIN - 1)
        for w in range(waited + 1, NWIN):
            wait_scatter(w % P)

        plsc.subcore_barrier()
        for t in range(NCH):
            sl = pl.ds(r0 + t * CH, CH)
            pltpu.async_copy(acc.at[sl], out_hbm.at[c, sl], csem)
        for t in range(NCH):
            pltpu.make_async_copy(acc.at[pl.ds(r0, CH)],
                                  out_hbm.at[c, pl.ds(r0, CH)], csem).wait()

    return spmm_kernel


def _blend(partial, c1, v):
    N, D = v.shape
    B = 1000

    def body(p_ref, c1_ref, v_ref, o_ref):
        o_ref[...] = (p_ref[0] + p_ref[1]) * c1_ref[...] + v_ref[...]

    return pl.pallas_call(
        body,
        grid=(N // B,),
        in_specs=[
            pl.BlockSpec((_NC, B, D), lambda i: (0, i, 0)),
            pl.BlockSpec((B, D), lambda i: (i, 0)),
            pl.BlockSpec((B, D), lambda i: (i, 0)),
        ],
        out_specs=pl.BlockSpec((B, D), lambda i: (i, 0)),
        out_shape=jax.ShapeDtypeStruct((N, D), jnp.float32),
    )(partial, c1, v)


def kernel(x, edge_index):
    N, D = x.shape
    E = edge_index.shape[1]

    row = edge_index[0]
    col = edge_index[1]

    deg_kernel, npad_deg = _make_deg(N, E)
    pd = deg_kernel(row)
    deg = pd[:N] + pd[npad_deg:npad_deg + N]
    _W = 80
    eint = jnp.concatenate(
        [col.reshape(-1, _W), row.reshape(-1, _W)], axis=1).reshape(-1)
    s = jnp.where(deg > 0.0, lax.rsqrt(deg), 0.0)

    u = x * s[:, None]
    v = _ALPHA * u
    c1 = jnp.broadcast_to((_BETA * (s * s))[:, None], (N, D))
    c1_last = jnp.broadcast_to(((_BETA / _GAMMA) * s)[:, None], (N, D))
    v_last = (_ALPHA / _GAMMA) * x

    spmm = _make_spmm(N, D, E)
    for _ in range(_K - 1):
        u = _blend(spmm(u, eint), c1, v)
    return _blend(spmm(u, eint), c1_last, v_last)


# speedup vs baseline: 1.0090x; 1.0090x over previous
import functools

import jax
import jax.numpy as jnp
from jax import lax
from jax.experimental import pallas as pl
from jax.experimental.pallas import tpu as pltpu
from jax.experimental.pallas import tpu_sc as plsc

_ALPHA = 0.1
_BETA = 0.9
_K = 10
_GAMMA = _BETA ** _K + _ALPHA * float(sum(_BETA ** i for i in range(_K)))

_NC = 2
_NS = 16
_NW = _NC * _NS


@functools.lru_cache(maxsize=None)
def _make_deg(N, E):
    EPW = E // _NW
    W = 80
    NWIN = EPW // W
    NPAD = ((N + 128 * _NS - 1) // (128 * _NS)) * (128 * _NS)
    FPT = NPAD // _NS
    mesh = plsc.VectorSubcoreMesh(core_axis_name="c", subcore_axis_name="s")

    @functools.partial(
        pl.kernel,
        out_type=jax.ShapeDtypeStruct((_NC * NPAD,), jnp.float32),
        mesh=mesh,
        scratch_types=(
            [pltpu.VMEM((W,), jnp.int32) for _ in range(4)]
            + [pltpu.VMEM((W,), jnp.float32),
               pltpu.VMEM((FPT,), jnp.float32),
               pltpu.VMEM_SHARED((NPAD,), jnp.float32)]
            + [pltpu.SemaphoreType.DMA for _ in range(9)]
        ),
    )
    def deg_kernel(row_hbm, out_hbm, *sc):
        P = 4
        idxw = sc[0:P]
        oneb = sc[P]
        zb = sc[P + 1]
        dacc = sc[P + 2]
        isem = sc[P + 3:2 * P + 3]
        ssem = sc[2 * P + 3:3 * P + 3]
        csem = sc[3 * P + 3]

        c = lax.axis_index("c")
        s = lax.axis_index("s")
        one = jnp.ones((16,), jnp.float32)
        zero = jnp.zeros((16,), jnp.float32)
        for q in range(W // 16):
            oneb[pl.ds(q * 16, 16)] = one
        for q in range(FPT // 16):
            zb[pl.ds(q * 16, 16)] = zero
        pltpu.sync_copy(zb, dacc.at[pl.ds(s * FPT, FPT)])
        plsc.subcore_barrier()
        base = (c * _NS + s) * EPW

        def stage(t, p):
            pltpu.async_copy(row_hbm.at[pl.ds(base + t * W, W)], idxw[p],
                             isem[p])

        def fire_scatter(p):
            pltpu.make_async_copy(row_hbm.at[pl.ds(base, W)], idxw[p],
                                  isem[p]).wait()
            pltpu.async_copy(oneb, dacc.at[idxw[p]], ssem[p], add=True)

        def wait_scatter(p):
            pltpu.make_async_copy(oneb, dacc.at[idxw[p]], ssem[p]).wait()

        stage(0, 0)

        def chunk(jj, carry):
            for p_off in range(P):
                t = 1 + jj * P + p_off
                p_a = (1 + p_off) % P
                p_b = p_off % P

                @pl.when(jnp.logical_and(t >= P, t - P < NWIN))
                def _():
                    wait_scatter(p_a)

                @pl.when(t < NWIN)
                def _():
                    stage(t, p_a)

                @pl.when(t - 1 < NWIN)
                def _():
                    fire_scatter(p_b)
            return carry

        nchunks = NWIN // P
        lax.fori_loop(0, nchunks, chunk, 0)
        t_last = nchunks * P
        for t in range(t_last + 1, NWIN + 1):
            wait_scatter((t - P) % P)
            fire_scatter((t - 1) % P)
        w_hi = (NWIN - P) if NWIN > t_last else (t_last - P)
        for w in range(w_hi + 1, NWIN):
            wait_scatter(w % P)

        plsc.subcore_barrier()
        pltpu.sync_copy(dacc.at[pl.ds(s * FPT, FPT)],
                        out_hbm.at[pl.ds(c * NPAD + s * FPT, FPT)])

    return deg_kernel, NPAD


@functools.lru_cache(maxsize=None)
def _make_spmm(N, D, E):
    EPW = E // _NW
    W = 80
    NWIN = EPW // W
    NPAD = ((N + 32 * _NS - 1) // (32 * _NS)) * (32 * _NS)
    RPT = NPAD // _NS
    CH = 32
    NCH = RPT // CH
    P = 4
    mesh = plsc.VectorSubcoreMesh(core_axis_name="c", subcore_axis_name="s")

    scratch = (
        [pltpu.VMEM((W,), jnp.int32) for _ in range(P)]
        + [pltpu.VMEM((W,), jnp.int32) for _ in range(P)]
        + [pltpu.VMEM((W, D), jnp.float32) for _ in range(P)]
        + [pltpu.VMEM((CH, D), jnp.float32)]
        + [pltpu.VMEM_SHARED((NPAD, D), jnp.float32)]
        + [pltpu.SemaphoreType.DMA for _ in range(3 * P + 1)]
    )

    @functools.partial(
        pl.kernel,
        out_type=jax.ShapeDtypeStruct((_NC, NPAD, D), jnp.float32),
        mesh=mesh,
        scratch_types=scratch,
    )
    def spmm_kernel(u_hbm, row_hbm, col_hbm, out_hbm, *sc):
        colw = sc[0:P]
        roww = sc[P:2 * P]
        gbuf = sc[2 * P:3 * P]
        zbuf = sc[3 * P]
        acc = sc[3 * P + 1]
        isem = sc[3 * P + 2:4 * P + 2]
        gsem = sc[4 * P + 2:5 * P + 2]
        ssem = sc[5 * P + 2:6 * P + 2]
        csem = sc[6 * P + 2]

        c = lax.axis_index("c")
        s = lax.axis_index("s")
        zero = jnp.zeros((16,), jnp.float32)
        for r in range(CH):
            for q in range(D // 16):
                zbuf[r, pl.ds(q * 16, 16)] = zero
        r0 = s * RPT
        for t in range(NCH):
            pltpu.async_copy(zbuf, acc.at[pl.ds(r0 + t * CH, CH)], csem)
        base = (c * _NS + s) * EPW
        for t in range(NCH):
            pltpu.make_async_copy(zbuf, acc.at[pl.ds(r0, CH)], csem).wait()
        plsc.subcore_barrier()

        def stage_idx(t, p):
            off = base + t * W
            pltpu.async_copy(col_hbm.at[pl.ds(off, W)], colw[p], isem[p])
            pltpu.async_copy(row_hbm.at[pl.ds(off, W)], roww[p], isem[p])

        def fire_gather(t, p):
            pltpu.make_async_copy(col_hbm.at[pl.ds(base, W)], colw[p],
                                  isem[p]).wait()
            pltpu.make_async_copy(row_hbm.at[pl.ds(base, W)], roww[p],
                                  isem[p]).wait()
            pltpu.async_copy(u_hbm.at[colw[p]], gbuf[p], gsem[p])

        def fire_scatter(p):
            pltpu.make_async_copy(u_hbm.at[colw[p]], gbuf[p],
                                  gsem[p]).wait()
            pltpu.async_copy(gbuf[p], acc.at[roww[p]], ssem[p], add=True)

        def wait_scatter(p):
            pltpu.make_async_copy(gbuf[p], acc.at[roww[p]], ssem[p]).wait()

        stage_idx(0, 0)
        stage_idx(1, 1)
        fire_gather(0, 0)

        def chunk(jj, carry):
            for p_off in range(P):
                t = 2 + jj * P + p_off

                @pl.when(jnp.logical_and(t >= 4, t - 4 < NWIN))
                def _():
                    wait_scatter((2 + p_off) % P)

                @pl.when(t < NWIN)
                def _():
                    stage_idx(t, (2 + p_off) % P)

                @pl.when(t <= NWIN)
                def _():
                    fire_gather(t - 1, (1 + p_off) % P)

                @pl.when(t - 2 < NWIN)
                def _():
                    fire_scatter(p_off % P)
            return carry

        nchunks = (NWIN - 1 + P - 1) // P
        lax.fori_loop(0, nchunks, chunk, 0)
        t_last = 2 + nchunks * P - 1
        done = min(nchunks * P, NWIN)
        for t in range(done, NWIN):
            fire_scatter(t % P)
        waited = min(t_last - 4, NWIN - 1)
        for w in range(waited + 1, NWIN):
            wait_scatter(w % P)

        plsc.subcore_barrier()
        for t in range(NCH):
            sl = pl.ds(r0 + t * CH, CH)
            pltpu.async_copy(acc.at[sl], out_hbm.at[c, sl], csem)
        for t in range(NCH):
            pltpu.make_async_copy(acc.at[pl.ds(r0, CH)],
                                  out_hbm.at[c, pl.ds(r0, CH)], csem).wait()

    return spmm_kernel


def _blend(partial, c1, v):
    N, D = v.shape
    B = 1000

    def body(p_ref, c1_ref, v_ref, o_ref):
        o_ref[...] = (p_ref[0] + p_ref[1]) * c1_ref[...] + v_ref[...]

    return pl.pallas_call(
        body,
        grid=(N // B,),
        in_specs=[
            pl.BlockSpec((_NC, B, D), lambda i: (0, i, 0)),
            pl.BlockSpec((B, 1), lambda i: (i, 0)),
            pl.BlockSpec((B, D), lambda i: (i, 0)),
        ],
        out_specs=pl.BlockSpec((B, D), lambda i: (i, 0)),
        out_shape=jax.ShapeDtypeStruct((N, D), jnp.float32),
    )(partial, c1, v)


def kernel(x, edge_index):
    N, D = x.shape
    E = edge_index.shape[1]

    row = edge_index[0]
    col = edge_index[1]

    deg_kernel, npad_deg = _make_deg(N, E)
    pd = deg_kernel(row)
    deg = pd[:N] + pd[npad_deg:npad_deg + N]
    s = jnp.where(deg > 0.0, lax.rsqrt(deg), 0.0)

    u = x * s[:, None]
    v = _ALPHA * u
    c1 = (_BETA * (s * s))[:, None]
    c1_last = ((_BETA / _GAMMA) * s)[:, None]
    v_last = (_ALPHA / _GAMMA) * x

    spmm = _make_spmm(N, D, E)
    for _ in range(_K - 1):
        u = _blend(spmm(u, row, col), c1, v)
    return _blend(spmm(u, row, col), c1_last, v_last)
